# Initial kernel scaffold; baseline (speedup 1.0000x reference)
import jax
import jax.numpy as jnp
from jax.experimental import pallas as pl


def kernel(input_1, input_2, embed, W, U, b, Wd, bd):
    raise NotImplementedError("write your pallas kernel here")


# speedup vs baseline: 2.6108x; 2.6108x over previous
import functools

import jax
import jax.numpy as jnp
from jax import lax
from jax.experimental import pallas as pl
from jax.experimental.pallas import tpu as pltpu
from jax.experimental.pallas import tpu_sc as plsc

DP = 64
HP = 128
H = 64


def _make_sc_gather(n_rows, d, n_workers=32, k=8):
    sg_rows = k * 128
    n_sg = n_rows // sg_rows
    assert n_sg * sg_rows == n_rows
    n_outer = (n_sg + n_workers - 1) // n_workers

    mesh = plsc.VectorSubcoreMesh(core_axis_name="c", subcore_axis_name="s")

    @functools.partial(
        pl.kernel,
        mesh=mesh,
        compiler_params=pltpu.CompilerParams(use_tc_tiling_on_sc=False),
        out_type=jax.ShapeDtypeStruct((n_rows, d), jnp.float32),
        scratch_types=[
            pltpu.VMEM((k, 128), jnp.int32),
            pltpu.VMEM((sg_rows, d), jnp.float32),
            pltpu.SemaphoreType.DMA,
        ],
    )
    def gather(emb_hbm, idx_hbm, out_hbm, idx_v, rows_v, sem):
        wid = lax.axis_index("s") * 2 + lax.axis_index("c")

        def body(j, carry):
            sg = j * n_workers + wid

            @pl.when(sg < n_sg)
            def _():
                ir0 = pl.multiple_of(sg * k, 8)
                pltpu.sync_copy(idx_hbm.at[pl.ds(ir0, k)], idx_v)
                cps = [
                    pltpu.async_copy(
                        emb_hbm.at[idx_v.at[j2]],
                        rows_v.at[pl.ds(j2 * 128, 128)],
                        sem,
                    )
                    for j2 in range(k)
                ]
                for cp in cps:
                    cp.wait()
                r0 = pl.multiple_of(sg * sg_rows, 8)
                pltpu.sync_copy(rows_v, out_hbm.at[pl.ds(r0, sg_rows)])

            return carry

        lax.fori_loop(0, n_outer, body, 0)

    return gather


def _make_lstm_call(b2, t_total, bb2, tc, interpret=False):
    nb = b2 // bb2
    nt = t_total // tc
    half = bb2 // 2

    def body(e_ref, w_ref, u_ref, b_ref, wd_ref, bd_ref, out_ref, h_ref, c_ref):
        t_idx = pl.program_id(1)

        @pl.when(t_idx == 0)
        def _():
            h_ref[...] = jnp.zeros((bb2, HP), jnp.float32)
            c_ref[...] = jnp.zeros((bb2, HP), jnp.float32)

        w = w_ref[...]
        u = u_ref[...]
        bias = b_ref[...]

        def step(tt, hc):
            h, c = hc
            x = e_ref[tt]
            z = (jnp.dot(x, w, preferred_element_type=jnp.float32)
                 + jnp.dot(h, u, preferred_element_type=jnp.float32)
                 + bias)
            gi = jax.nn.sigmoid(z[:, :HP])
            gf = jax.nn.sigmoid(z[:, HP:2 * HP])
            gg = jnp.tanh(z[:, 2 * HP:3 * HP])
            go = jax.nn.sigmoid(z[:, 3 * HP:])
            c = gf * c + gi * gg
            h = go * jnp.tanh(c)
            return (h, c)

        h, c = lax.fori_loop(0, tc, step, (h_ref[...], c_ref[...]))
        h_ref[...] = h
        c_ref[...] = c

        @pl.when(t_idx == nt - 1)
        def _():
            merged = jnp.concatenate([h[:half, :H], h[half:, :H]], axis=1)
            logits = (jnp.dot(merged, wd_ref[...], preferred_element_type=jnp.float32)
                      + bd_ref[...])
            m = jnp.max(logits, axis=1, keepdims=True)
            p = jnp.exp(logits - m)
            out_ref[...] = p / jnp.sum(p, axis=1, keepdims=True)

    return pl.pallas_call(
        body,
        grid=(nb, nt),
        in_specs=[
            pl.BlockSpec((tc, bb2, DP), lambda i, t: (t, i, 0)),
            pl.BlockSpec((DP, 4 * HP), lambda i, t: (0, 0)),
            pl.BlockSpec((HP, 4 * HP), lambda i, t: (0, 0)),
            pl.BlockSpec((1, 4 * HP), lambda i, t: (0, 0)),
            pl.BlockSpec((HP, 3), lambda i, t: (0, 0)),
            pl.BlockSpec((1, 3), lambda i, t: (0, 0)),
        ],
        out_specs=pl.BlockSpec((half, 3), lambda i, t: (i, 0)),
        out_shape=jax.ShapeDtypeStruct((b2 // 2, 3), jnp.float32),
        scratch_shapes=[
            pltpu.VMEM((bb2, HP), jnp.float32),
            pltpu.VMEM((bb2, HP), jnp.float32),
        ],
        interpret=interpret,
    )


def _prep_weights(W, U, b, Wd, bd):
    d, fh = W.shape
    h = U.shape[0]
    g = fh // 4
    Wp = jnp.pad(W.reshape(d, 4, g), ((0, DP - d), (0, 0), (0, HP - g)))
    Wp = Wp.reshape(DP, 4 * HP)
    Up = jnp.pad(U.reshape(h, 4, g), ((0, HP - h), (0, 0), (0, HP - g)))
    Up = Up.reshape(HP, 4 * HP)
    bp = jnp.pad(b.reshape(4, g), ((0, 0), (0, HP - g))).reshape(1, 4 * HP)
    return Wp, Up, bp, Wd, bd.reshape(1, -1)


def _build_indices(input_1, input_2, nb, half):
    t = input_1.shape[1]
    i1 = input_1.T.reshape(t, nb, half)
    i2 = input_2.T.reshape(t, nb, half)
    return jnp.stack([i1, i2], axis=2).reshape(-1, 128).astype(jnp.int32)


def kernel(input_1, input_2, embed, W, U, b, Wd, bd):
    bsz, t_total = input_1.shape
    half = 256
    bb2 = 2 * half
    nb = bsz // half
    n_rows = 2 * bsz * t_total

    emb_pad = jnp.pad(embed, ((0, 0), (0, DP - embed.shape[1])))
    Wp, Up, bp, Wdp, bdp = _prep_weights(W, U, b, Wd, bd)
    idx = _build_indices(input_1, input_2, nb, half)

    e = _make_sc_gather(n_rows, DP)(emb_pad, idx)
    e = e.reshape(t_total, 2 * bsz, DP)

    out = _make_lstm_call(2 * bsz, t_total, bb2, 8)(e, Wp, Up, bp, Wdp, bdp)
    return out


# speedup vs baseline: 3.9094x; 1.4974x over previous
import functools

import jax
import jax.numpy as jnp
from jax import lax
from jax.experimental import pallas as pl
from jax.experimental.pallas import tpu as pltpu
from jax.experimental.pallas import tpu_sc as plsc

DP = 64
HP = 128
H = 64


def _make_sc_gather(n_rows, d, n_workers=32, k=8):
    sg_rows = k * 128
    n_sg = n_rows // sg_rows
    assert n_sg * sg_rows == n_rows
    n_outer = (n_sg + n_workers - 1) // n_workers

    mesh = plsc.VectorSubcoreMesh(core_axis_name="c", subcore_axis_name="s")

    @functools.partial(
        pl.kernel,
        mesh=mesh,
        compiler_params=pltpu.CompilerParams(use_tc_tiling_on_sc=False),
        out_type=jax.ShapeDtypeStruct((n_rows, d), jnp.float32),
        scratch_types=[
            pltpu.VMEM((k, 128), jnp.int32),
            pltpu.VMEM((sg_rows, d), jnp.float32),
            pltpu.SemaphoreType.DMA,
        ],
    )
    def gather(emb_hbm, idx_hbm, out_hbm, idx_v, rows_v, sem):
        wid = lax.axis_index("s") * 2 + lax.axis_index("c")

        def body(j, carry):
            sg = j * n_workers + wid

            @pl.when(sg < n_sg)
            def _():
                ir0 = pl.multiple_of(sg * k, 8)
                pltpu.sync_copy(idx_hbm.at[pl.ds(ir0, k)], idx_v)
                cps = [
                    pltpu.async_copy(
                        emb_hbm.at[idx_v.at[j2]],
                        rows_v.at[pl.ds(j2 * 128, 128)],
                        sem,
                    )
                    for j2 in range(k)
                ]
                for cp in cps:
                    cp.wait()
                r0 = pl.multiple_of(sg * sg_rows, 8)
                pltpu.sync_copy(rows_v, out_hbm.at[pl.ds(r0, sg_rows)])

            return carry

        lax.fori_loop(0, n_outer, body, 0)

    return gather


def _make_lstm_call(b2, t_total, tc, interpret=False):
    nt = t_total // tc
    half = b2 // 2

    def body(e_ref, wt_ref, ut_ref, bt_ref, wdt_ref, bdt_ref, out_ref,
             h_ref, c_ref):
        t_idx = pl.program_id(0)

        @pl.when(t_idx == 0)
        def _():
            h_ref[...] = jnp.zeros((H, b2), jnp.float32)
            c_ref[...] = jnp.zeros((H, b2), jnp.float32)

        wt = wt_ref[...]
        ut = ut_ref[...]
        bt = bt_ref[...]

        def step(tt, hc):
            h, c = hc
            x = e_ref[tt]
            z = lax.dot_general(wt, x, (((1,), (1,)), ((), ())),
                                preferred_element_type=jnp.float32)
            z = z + jnp.dot(ut, h, preferred_element_type=jnp.float32) + bt
            gi = jax.nn.sigmoid(z[0 * H:1 * H])
            gf = jax.nn.sigmoid(z[1 * H:2 * H])
            gg = jnp.tanh(z[2 * H:3 * H])
            go = jax.nn.sigmoid(z[3 * H:4 * H])
            c = gf * c + gi * gg
            h = go * jnp.tanh(c)
            return (h, c)

        h, c = lax.fori_loop(0, tc, step, (h_ref[...], c_ref[...]))
        h_ref[...] = h
        c_ref[...] = c

        @pl.when(t_idx == nt - 1)
        def _():
            merged = jnp.concatenate([h[:, :half], h[:, half:]], axis=0)
            logits = (jnp.dot(wdt_ref[...], merged,
                              preferred_element_type=jnp.float32)
                      + bdt_ref[...])
            m = jnp.max(logits, axis=0, keepdims=True)
            p = jnp.exp(logits - m)
            out_ref[...] = p / jnp.sum(p, axis=0, keepdims=True)

    return pl.pallas_call(
        body,
        grid=(nt,),
        in_specs=[
            pl.BlockSpec((tc, b2, DP), lambda t: (t, 0, 0)),
            pl.BlockSpec((4 * H, DP), lambda t: (0, 0)),
            pl.BlockSpec((4 * H, H), lambda t: (0, 0)),
            pl.BlockSpec((4 * H, 1), lambda t: (0, 0)),
            pl.BlockSpec((8, 2 * H), lambda t: (0, 0)),
            pl.BlockSpec((8, 1), lambda t: (0, 0)),
        ],
        out_specs=pl.BlockSpec((8, half), lambda t: (0, 0)),
        out_shape=jax.ShapeDtypeStruct((8, half), jnp.float32),
        scratch_shapes=[
            pltpu.VMEM((H, b2), jnp.float32),
            pltpu.VMEM((H, b2), jnp.float32),
        ],
        interpret=interpret,
    )


def _prep_weights(W, U, b, Wd, bd):
    d = W.shape[0]
    Wt = jnp.pad(W, ((0, DP - d), (0, 0))).T
    Ut = U.T
    bt = b.reshape(-1, 1)
    Wdt = jnp.pad(Wd.T, ((0, 5), (0, 0)))
    bdt = jnp.concatenate([bd, jnp.full((5,), -1e30, bd.dtype)]).reshape(8, 1)
    return Wt, Ut, bt, Wdt, bdt


def _build_indices(input_1, input_2):
    return jnp.concatenate([input_1.T, input_2.T], axis=1).reshape(-1, 128).astype(jnp.int32)


def kernel(input_1, input_2, embed, W, U, b, Wd, bd):
    bsz, t_total = input_1.shape
    b2 = 2 * bsz
    n_rows = b2 * t_total

    emb_pad = jnp.pad(embed, ((0, 0), (0, DP - embed.shape[1])))
    Wt, Ut, bt, Wdt, bdt = _prep_weights(W, U, b, Wd, bd)
    idx = _build_indices(input_1, input_2)

    e = _make_sc_gather(n_rows, DP)(emb_pad, idx)
    e = e.reshape(t_total, b2, DP)

    out_t = _make_lstm_call(b2, t_total, 8)(e, Wt, Ut, bt, Wdt, bdt)
    return out_t[:3].T
